# baseline (speedup 1.0000x reference)
import functools

import numpy as np
import jax
import jax.numpy as jnp
from jax import lax
from jax.experimental import pallas as pl
from jax.experimental.pallas import tpu as pltpu
from jax.experimental.pallas import tpu_sc as plsc

N = 10000; E = 160000; DN = 128; DE = 16; H = 16
T = 3; ITERS = 3; NMOL = 1000; NRXN = 250; IN2 = 32; DH = 128
NC = 2; NS = 16; NW = NC * NS
NPAD = 10240
EPAD = 163840
ACCM = 1008
WD = 48
f32 = jnp.float32


def _sc_gather(table, idx3, trows, d, c_chunks, k_chunk, pack128=False):
    per = c_chunks * k_chunk
    mesh = plsc.VectorSubcoreMesh(core_axis_name="c", subcore_axis_name="s")
    if pack128:
        out_t = jax.ShapeDtypeStruct((NW, per, d), f32)
    else:
        out_t = jax.ShapeDtypeStruct((NW * per, d), f32)

    @functools.partial(
        pl.kernel,
        out_type=out_t,
        mesh=mesh,
        compiler_params=pltpu.CompilerParams(use_tc_tiling_on_sc=False),
        scratch_types=[
            pltpu.VMEM((c_chunks, k_chunk), jnp.int32),
            pltpu.VMEM((per, d), f32),
            pltpu.SemaphoreType.DMA,
        ],
    )
    def k(table_hbm, idx_hbm, out_hbm, idx_v, rows_v, sem):
        wid = (lax.axis_index("c") * np.int32(NS)
               + lax.axis_index("s")).astype(jnp.int32)
        pltpu.sync_copy(idx_hbm.at[wid], idx_v)

        wsz = 20 if c_chunks % 20 == 0 else min(8, c_chunks)

        def wave(wv, carry):
            descs = []
            for j in range(wsz):
                c = wv * np.int32(wsz) + np.int32(j)
                descs.append(pltpu.async_copy(
                    table_hbm.at[idx_v.at[c]],
                    rows_v.at[pl.ds(c * np.int32(k_chunk), k_chunk)],
                    sem,
                ))
            for dsc in descs:
                dsc.wait()
            return carry

        lax.fori_loop(jnp.int32(0), jnp.int32(c_chunks // wsz), wave,
                      jnp.int32(0))

        if pack128:
            pltpu.sync_copy(rows_v, out_hbm.at[wid])
        else:
            pltpu.sync_copy(rows_v,
                            out_hbm.at[pl.ds(wid * np.int32(per), per)])

    return k(table, idx3)


def _sc_scatter_add(vals, idx3, zeros, arows, d, c_chunks, k_chunk,
                    pack128=False):
    per = c_chunks * k_chunk
    rps = arows // NS
    prow = per * d // 128
    mesh = plsc.VectorSubcoreMesh(core_axis_name="c", subcore_axis_name="s")

    @functools.partial(
        pl.kernel,
        out_type=jax.ShapeDtypeStruct((NC, arows, d), f32),
        mesh=mesh,
        compiler_params=pltpu.CompilerParams(use_tc_tiling_on_sc=False),
        scratch_types=[
            pltpu.VMEM((c_chunks, k_chunk), jnp.int32),
            pltpu.VMEM((per, d), f32),
            pltpu.VMEM_SHARED((arows, d), f32),
            pltpu.SemaphoreType.DMA,
        ],
    )
    def k(vals_hbm, idx_hbm, zeros_hbm, out_hbm, idx_v, vals_v, acc, sem):
        cid = lax.axis_index("c").astype(jnp.int32)
        sid = lax.axis_index("s").astype(jnp.int32)
        wid = cid * np.int32(NS) + sid
        pltpu.sync_copy(zeros_hbm.at[pl.ds(sid * np.int32(rps), rps)],
                        acc.at[pl.ds(sid * np.int32(rps), rps)])
        pltpu.sync_copy(idx_hbm.at[wid], idx_v)
        if pack128:
            pltpu.sync_copy(vals_hbm.at[wid], vals_v)
        else:
            pltpu.sync_copy(vals_hbm.at[pl.ds(wid * np.int32(per), per)],
                            vals_v)
        plsc.subcore_barrier()

        wsz = 20 if c_chunks % 20 == 0 else min(8, c_chunks)

        def wave(wv, carry):
            descs = []
            for j in range(wsz):
                c = wv * np.int32(wsz) + np.int32(j)
                descs.append(pltpu.async_copy(
                    vals_v.at[pl.ds(c * np.int32(k_chunk), k_chunk)],
                    acc.at[idx_v.at[c]], sem, add=True))
            for dsc in descs:
                dsc.wait()
            return carry

        lax.fori_loop(jnp.int32(0), jnp.int32(c_chunks // wsz), wave,
                      jnp.int32(0))

        plsc.subcore_barrier()
        pltpu.sync_copy(acc.at[pl.ds(sid * np.int32(rps), rps)],
                        out_hbm.at[cid, pl.ds(sid * np.int32(rps), rps)])

    return k(vals, idx3, zeros)


def _proj(na, wp, bp):
    def body(na_ref, wp_ref, bp_ref, out_ref):
        y = jnp.maximum(
            jnp.dot(na_ref[...], wp_ref[...], preferred_element_type=f32)
            + bp_ref[...], 0.0)
        out_ref[...] = jnp.concatenate(
            [y, jnp.zeros((NPAD - N, H), f32)], axis=0)

    return pl.pallas_call(
        body,
        out_shape=jax.ShapeDtypeStruct((NPAD, H), f32),
    )(na, wp, bp)


_EB = 4096


def _ilv_t(both):
    nb2 = both.shape[0]

    def body(in_ref, out_ref):
        out_ref[0] = in_ref[0].T

    z32 = np.int32(0)
    return pl.pallas_call(
        body,
        grid=(nb2,),
        in_specs=[pl.BlockSpec((1, 8, _EB // 8), lambda i: (i, z32, z32))],
        out_specs=pl.BlockSpec((1, _EB // 8, 8), lambda i: (i, z32, z32)),
        out_shape=jax.ShapeDtypeStruct((nb2, _EB // 8, 8), f32),
    )(both)


def _msg(ea, xs128, r1, r2, w2, bm):
    pb = _EB * H // 128
    G = _EB // 8

    def body(ea_ref, xs_ref, r1_ref, r2_ref, w2_ref, bm_ref, out_ref):
        r1_ = r1_ref[...]
        r2_ = r2_ref[...]
        w2_ = w2_ref[...]
        bm_ = bm_ref[...]
        for j in range(8):
            xj = xs_ref[:, j * H:(j + 1) * H]
            eaj = ea_ref[j * G:(j + 1) * G, :]
            opj = (jnp.dot(xj, r1_, preferred_element_type=f32)
                   * jnp.dot(eaj, r2_, preferred_element_type=f32))
            out_ref[:, j * H:(j + 1) * H] = (
                jnp.dot(opj, w2_, preferred_element_type=f32)
                + jnp.dot(xj, bm_, preferred_element_type=f32))

    z32 = np.int32(0)
    nb = EPAD // _EB
    return pl.pallas_call(
        body,
        grid=(nb,),
        in_specs=[
            pl.BlockSpec((_EB, DE), lambda i: (i, z32)),
            pl.BlockSpec((pb, 128), lambda i: (i, z32)),
            pl.BlockSpec((H, H * H), lambda i: (z32, z32)),
            pl.BlockSpec((DE, H * H), lambda i: (z32, z32)),
            pl.BlockSpec((H * H, H), lambda i: (z32, z32)),
            pl.BlockSpec((H, H), lambda i: (z32, z32)),
        ],
        out_specs=pl.BlockSpec((pb, 128), lambda i: (i, z32)),
        out_shape=jax.ShapeDtypeStruct((EPAD * H // 128, 128), f32),
    )(ea, xs128, r1, r2, w2, bm)


def _gru(a0, a1, h, bnn, wih_t, whh_t, bih, bhh):
    def body(a0_ref, a1_ref, h_ref, bnn_ref, wih_ref, whh_ref, bih_ref,
             bhh_ref, out_ref):
        xa = jnp.maximum(a0_ref[...] + a1_ref[...] + bnn_ref[...], 0.0)
        h_ = h_ref[...]
        gi = jnp.dot(xa, wih_ref[...], preferred_element_type=f32) + bih_ref[...]
        gh = jnp.dot(h_, whh_ref[...], preferred_element_type=f32) + bhh_ref[...]
        r = jax.nn.sigmoid(gi[:, :H] + gh[:, :H])
        z = jax.nn.sigmoid(gi[:, H:2 * H] + gh[:, H:2 * H])
        n_ = jnp.tanh(gi[:, 2 * H:] + r * gh[:, 2 * H:])
        hn = (1.0 - z) * n_ + z * h_
        rows = lax.broadcasted_iota(jnp.int32, (NPAD, H), 0)
        out_ref[...] = jnp.where(rows < N, hn, 0.0)

    return pl.pallas_call(
        body,
        out_shape=jax.ShapeDtypeStruct((NPAD, H), f32),
    )(a0, a1, h, bnn, wih_t, whh_t, bih, bhh)


def _s2s_update(is_first, wa0, wa1, qprev, lc, wih_t, whh_t, bih, bhh):
    def body(wa0_ref, wa1_ref, q_ref, lc_ref, wih_ref, whh_ref, bih_ref,
             bhh_ref, qtab_ref, lco_ref):
        qp = q_ref[...]
        if is_first:
            qs = jnp.zeros((NMOL, 2 * IN2), f32)
        else:
            acc = wa0_ref[...] + wa1_ref[...]
            accn = acc[:NMOL]
            ro = accn[:, :IN2] / accn[:, IN2:IN2 + 1]
            qs = jnp.concatenate([qp, ro], axis=1)
        gates = (jnp.dot(qs, wih_ref[...], preferred_element_type=f32)
                 + bih_ref[...]
                 + jnp.dot(qp, whh_ref[...], preferred_element_type=f32)
                 + bhh_ref[...])
        i_ = jax.nn.sigmoid(gates[:, :IN2])
        ff = jax.nn.sigmoid(gates[:, IN2:2 * IN2])
        g_ = jnp.tanh(gates[:, 2 * IN2:3 * IN2])
        o_ = jax.nn.sigmoid(gates[:, 3 * IN2:])
        lcn = ff * lc_ref[...] + i_ * g_
        q = o_ * jnp.tanh(lcn)
        qtab_ref[...] = jnp.concatenate(
            [q, jnp.zeros((ACCM - NMOL, IN2), f32)], axis=0)
        lco_ref[...] = lcn

    return pl.pallas_call(
        body,
        out_shape=[jax.ShapeDtypeStruct((ACCM, IN2), f32),
                   jax.ShapeDtypeStruct((NMOL, IN2), f32)],
    )(wa0, wa1, qprev, lc, wih_t, whh_t, bih, bhh)


def _ew(x, x0, qg):
    def body(x_ref, x0_ref, qg_ref, out_ref):
        nag = jnp.concatenate([x_ref[...], x0_ref[...]], axis=1)
        e = jnp.sum(nag * qg_ref[...], axis=1, keepdims=True)
        gmax = jnp.max(e)
        ex = jnp.exp(e - gmax)
        out_ref[...] = jnp.concatenate(
            [ex * nag, ex, jnp.zeros((NPAD, WD - IN2 - 1), f32)], axis=1)

    return pl.pallas_call(
        body,
        out_shape=jax.ShapeDtypeStruct((NPAD, WD), f32),
    )(x, x0, qg)


def _final(wa0, wa1, qprev, wsp, bsp, prelu, rxn_bc):
    def body(wa0_ref, wa1_ref, q_ref, wsp_ref, bsp_ref, pr_ref, rxn_ref,
             out_ref):
        acc = wa0_ref[...] + wa1_ref[...]
        accn = acc[:NMOL]
        ro = accn[:, :IN2] / accn[:, IN2:IN2 + 1]
        qs = jnp.concatenate([q_ref[...], ro], axis=1)
        rf = jnp.dot(qs, wsp_ref[...], preferred_element_type=f32) + bsp_ref[...]
        rf = jnp.where(rf >= jnp.float32(0.0), rf, pr_ref[...] * rf)
        rids = rxn_ref[...][0:1, :]
        rr = lax.broadcasted_iota(jnp.int32, (NRXN, NMOL), 0).astype(f32)
        mask = jnp.where(rids == rr, jnp.float32(1.0), jnp.float32(0.0))
        out_ref[...] = jnp.dot(mask, rf, preferred_element_type=f32)

    return pl.pallas_call(
        body,
        out_shape=jax.ShapeDtypeStruct((NRXN, DH), f32),
    )(wa0, wa1, qprev, wsp, bsp, prelu, rxn_bc)


_J = np.arange(H * H)
_R1_NP = (_J[None, :] // H == np.arange(H)[:, None]).astype(np.float32)
_R2_NP = (_J[None, :] % H == np.arange(DE)[:, None]).astype(np.float32)


def kernel(node_attr, edge_attr, W_proj, b_proj, W_bond, b_bond, b_nn,
           gru_Wih, gru_Whh, gru_bih, gru_bhh,
           lstm_Wih, lstm_Whh, lstm_bih, lstm_bhh,
           W_sp, b_sp, prelu_a,
           edge_index, mol_ids, rxn_ids):
    src = edge_index[0].astype(jnp.int32)
    dst = edge_index[1].astype(jnp.int32)
    epad = jnp.full((EPAD - E,), N, jnp.int32)

    nb = EPAD // _EB
    both = jnp.concatenate(
        [jnp.concatenate([src, epad]), jnp.concatenate([dst, epad])]
    ).astype(f32).reshape(2 * nb, 8, _EB // 8)
    both_t = _ilv_t(both).astype(jnp.int32).reshape(2, NW, 40, 128)
    src_p = both_t[0]
    dst_p = both_t[1]
    mol32 = mol_ids.astype(jnp.int32)
    mol_p = jnp.concatenate(
        [mol32, jnp.full((NPAD - N,), NMOL, jnp.int32)]).reshape(NW, 4, 80)
    ea = edge_attr.astype(f32)

    w2 = W_bond.astype(f32).reshape(DE, H, H).transpose(1, 0, 2).reshape(H * H, H)
    bm = b_bond.astype(f32).reshape(H, H)
    bp = b_proj.astype(f32).reshape(1, H)
    bnn = b_nn.astype(f32).reshape(1, H)
    g_wih_t = gru_Wih.astype(f32).T
    g_whh_t = gru_Whh.astype(f32).T
    g_bih = gru_bih.astype(f32).reshape(1, 3 * H)
    g_bhh = gru_bhh.astype(f32).reshape(1, 3 * H)
    l_wih_t = lstm_Wih.astype(f32).T
    l_whh_t = lstm_Whh.astype(f32).T
    l_bih = lstm_bih.astype(f32).reshape(1, 4 * IN2)
    l_bhh = lstm_bhh.astype(f32).reshape(1, 4 * IN2)
    wsp = W_sp.astype(f32)
    bsp = b_sp.astype(f32).reshape(1, DH)
    prelu = prelu_a.astype(f32).reshape(1, 1)
    rxn_bc = jnp.broadcast_to(
        rxn_ids.astype(f32).reshape(1, NMOL), (8, NMOL))

    zeros_n = jnp.zeros((NPAD, H), f32)
    zeros_m = jnp.zeros((ACCM, WD), f32)
    r1 = jnp.asarray(_R1_NP)
    r2 = jnp.asarray(_R2_NP)

    x0 = _proj(node_attr.astype(f32), W_proj.astype(f32), bp)
    x = x0

    for _ in range(T):
        xs3 = _sc_gather(x, src_p, NPAD, H, 40, 128, pack128=True)
        msg128 = _msg(ea, xs3.reshape(EPAD * H // 128, 128), r1, r2, w2, bm)
        acc2 = _sc_scatter_add(msg128.reshape(NW, EPAD // NW, H), dst_p,
                               zeros_n, NPAD, H, 40, 128, pack128=True)
        x = _gru(acc2[0], acc2[1], x, bnn, g_wih_t, g_whh_t, g_bih, g_bhh)

    qprev = jnp.zeros((NMOL, IN2), f32)
    lc = jnp.zeros((NMOL, IN2), f32)
    wa0 = zeros_m
    wa1 = zeros_m
    for i in range(ITERS):
        qtab, lc = _s2s_update(i == 0, wa0, wa1, qprev, lc,
                               l_wih_t, l_whh_t, l_bih, l_bhh)
        qprev = qtab[:NMOL]
        qg = _sc_gather(qtab, mol_p, ACCM, IN2, 4, 80)
        w = _ew(x, x0, qg)
        qg = _sc_gather(qtab, mol_p, ACCM, IN2, 4, 80)
        w = _ew(x, x0, qg)
        wacc = _sc_scatter_add(w, mol_p, zeros_m, ACCM, WD, 4, 80)
        wa0 = wacc[0]
        wa1 = wacc[1]

    return _final(wa0, wa1, qprev, wsp, bsp, prelu, rxn_bc)


# speedup vs baseline: 1.1478x; 1.1478x over previous
import functools

import numpy as np
import jax
import jax.numpy as jnp
from jax import lax
from jax.experimental import pallas as pl
from jax.experimental.pallas import tpu as pltpu
from jax.experimental.pallas import tpu_sc as plsc

N = 10000; E = 160000; DN = 128; DE = 16; H = 16
T = 3; ITERS = 3; NMOL = 1000; NRXN = 250; IN2 = 32; DH = 128
NC = 2; NS = 16; NW = NC * NS
NPAD = 10240
EPAD = 163840
ACCM = 1008
WD = 48
f32 = jnp.float32


def _sc_gather(table, idx3, trows, d, c_chunks, k_chunk, pack128=False):
    per = c_chunks * k_chunk
    mesh = plsc.VectorSubcoreMesh(core_axis_name="c", subcore_axis_name="s")
    if pack128:
        out_t = jax.ShapeDtypeStruct((NW, per, d), f32)
    else:
        out_t = jax.ShapeDtypeStruct((NW * per, d), f32)

    @functools.partial(
        pl.kernel,
        out_type=out_t,
        mesh=mesh,
        compiler_params=pltpu.CompilerParams(use_tc_tiling_on_sc=False),
        scratch_types=[
            pltpu.VMEM((c_chunks, k_chunk), jnp.int32),
            pltpu.VMEM((per, d), f32),
            pltpu.SemaphoreType.DMA,
        ],
    )
    def k(table_hbm, idx_hbm, out_hbm, idx_v, rows_v, sem):
        wid = (lax.axis_index("c") * np.int32(NS)
               + lax.axis_index("s")).astype(jnp.int32)
        pltpu.sync_copy(idx_hbm.at[wid], idx_v)

        wsz = 20 if c_chunks % 20 == 0 else min(8, c_chunks)

        def wave(wv, carry):
            descs = []
            for j in range(wsz):
                c = wv * np.int32(wsz) + np.int32(j)
                descs.append(pltpu.async_copy(
                    table_hbm.at[idx_v.at[c]],
                    rows_v.at[pl.ds(c * np.int32(k_chunk), k_chunk)],
                    sem,
                ))
            for dsc in descs:
                dsc.wait()
            return carry

        lax.fori_loop(jnp.int32(0), jnp.int32(c_chunks // wsz), wave,
                      jnp.int32(0))

        if pack128:
            pltpu.sync_copy(rows_v, out_hbm.at[wid])
        else:
            pltpu.sync_copy(rows_v,
                            out_hbm.at[pl.ds(wid * np.int32(per), per)])

    return k(table, idx3)


def _sc_scatter_add(vals, idx3, zeros, arows, d, c_chunks, k_chunk,
                    pack128=False):
    per = c_chunks * k_chunk
    rps = arows // NS
    prow = per * d // 128
    mesh = plsc.VectorSubcoreMesh(core_axis_name="c", subcore_axis_name="s")

    @functools.partial(
        pl.kernel,
        out_type=jax.ShapeDtypeStruct((NC, arows, d), f32),
        mesh=mesh,
        compiler_params=pltpu.CompilerParams(use_tc_tiling_on_sc=False),
        scratch_types=[
            pltpu.VMEM((c_chunks, k_chunk), jnp.int32),
            pltpu.VMEM((per, d), f32),
            pltpu.VMEM_SHARED((arows, d), f32),
            pltpu.SemaphoreType.DMA,
        ],
    )
    def k(vals_hbm, idx_hbm, zeros_hbm, out_hbm, idx_v, vals_v, acc, sem):
        cid = lax.axis_index("c").astype(jnp.int32)
        sid = lax.axis_index("s").astype(jnp.int32)
        wid = cid * np.int32(NS) + sid
        pltpu.sync_copy(zeros_hbm.at[pl.ds(sid * np.int32(rps), rps)],
                        acc.at[pl.ds(sid * np.int32(rps), rps)])
        pltpu.sync_copy(idx_hbm.at[wid], idx_v)
        if pack128:
            pltpu.sync_copy(vals_hbm.at[wid], vals_v)
        else:
            pltpu.sync_copy(vals_hbm.at[pl.ds(wid * np.int32(per), per)],
                            vals_v)
        plsc.subcore_barrier()

        wsz = 20 if c_chunks % 20 == 0 else min(8, c_chunks)

        def wave(wv, carry):
            descs = []
            for j in range(wsz):
                c = wv * np.int32(wsz) + np.int32(j)
                descs.append(pltpu.async_copy(
                    vals_v.at[pl.ds(c * np.int32(k_chunk), k_chunk)],
                    acc.at[idx_v.at[c]], sem, add=True))
            for dsc in descs:
                dsc.wait()
            return carry

        lax.fori_loop(jnp.int32(0), jnp.int32(c_chunks // wsz), wave,
                      jnp.int32(0))

        plsc.subcore_barrier()
        pltpu.sync_copy(acc.at[pl.ds(sid * np.int32(rps), rps)],
                        out_hbm.at[cid, pl.ds(sid * np.int32(rps), rps)])

    return k(vals, idx3, zeros)


def _proj(na, wp, bp):
    def body(na_ref, wp_ref, bp_ref, out_ref):
        y = jnp.maximum(
            jnp.dot(na_ref[...], wp_ref[...], preferred_element_type=f32)
            + bp_ref[...], 0.0)
        out_ref[...] = jnp.concatenate(
            [y, jnp.zeros((NPAD - N, H), f32)], axis=0)

    return pl.pallas_call(
        body,
        out_shape=jax.ShapeDtypeStruct((NPAD, H), f32),
    )(na, wp, bp)


_EB = 8192


def _msg(ea, xs128, r1, r2, w2, bm):
    pb = _EB * H // 128
    G = _EB // 8

    def body(ea_ref, xs_ref, r1_ref, r2_ref, w2_ref, bm_ref, out_ref):
        r1_ = r1_ref[...]
        r2_ = r2_ref[...]
        w2_ = w2_ref[...]
        bm_ = bm_ref[...]
        for j in range(8):
            xj = xs_ref[:, j * H:(j + 1) * H]
            eaj = ea_ref[j * G:(j + 1) * G, :]
            opj = (jnp.dot(xj, r1_, preferred_element_type=f32)
                   * jnp.dot(eaj, r2_, preferred_element_type=f32))
            out_ref[:, j * H:(j + 1) * H] = (
                jnp.dot(opj, w2_, preferred_element_type=f32)
                + jnp.dot(xj, bm_, preferred_element_type=f32))

    z32 = np.int32(0)
    nb = EPAD // _EB
    return pl.pallas_call(
        body,
        grid=(nb,),
        in_specs=[
            pl.BlockSpec((_EB, DE), lambda i: (i, z32)),
            pl.BlockSpec((pb, 128), lambda i: (i, z32)),
            pl.BlockSpec((H, H * H), lambda i: (z32, z32)),
            pl.BlockSpec((DE, H * H), lambda i: (z32, z32)),
            pl.BlockSpec((H * H, H), lambda i: (z32, z32)),
            pl.BlockSpec((H, H), lambda i: (z32, z32)),
        ],
        out_specs=pl.BlockSpec((pb, 128), lambda i: (i, z32)),
        out_shape=jax.ShapeDtypeStruct((EPAD * H // 128, 128), f32),
    )(ea, xs128, r1, r2, w2, bm)


def _gru(a0, a1, h, bnn, wih_t, whh_t, bih, bhh):
    def body(a0_ref, a1_ref, h_ref, bnn_ref, wih_ref, whh_ref, bih_ref,
             bhh_ref, out_ref):
        xa = jnp.maximum(a0_ref[...] + a1_ref[...] + bnn_ref[...], 0.0)
        h_ = h_ref[...]
        gi = jnp.dot(xa, wih_ref[...], preferred_element_type=f32) + bih_ref[...]
        gh = jnp.dot(h_, whh_ref[...], preferred_element_type=f32) + bhh_ref[...]
        r = jax.nn.sigmoid(gi[:, :H] + gh[:, :H])
        z = jax.nn.sigmoid(gi[:, H:2 * H] + gh[:, H:2 * H])
        n_ = jnp.tanh(gi[:, 2 * H:] + r * gh[:, 2 * H:])
        hn = (1.0 - z) * n_ + z * h_
        rows = lax.broadcasted_iota(jnp.int32, (NPAD, H), 0)
        out_ref[...] = jnp.where(rows < N, hn, 0.0)

    return pl.pallas_call(
        body,
        out_shape=jax.ShapeDtypeStruct((NPAD, H), f32),
    )(a0, a1, h, bnn, wih_t, whh_t, bih, bhh)


def _s2s_update(is_first, wa0, wa1, qprev, lc, wih_t, whh_t, bih, bhh):
    def body(wa0_ref, wa1_ref, q_ref, lc_ref, wih_ref, whh_ref, bih_ref,
             bhh_ref, qtab_ref, lco_ref):
        qp = q_ref[...]
        if is_first:
            qs = jnp.zeros((NMOL, 2 * IN2), f32)
        else:
            acc = wa0_ref[...] + wa1_ref[...]
            accn = acc[:NMOL]
            ro = accn[:, :IN2] / accn[:, IN2:IN2 + 1]
            qs = jnp.concatenate([qp, ro], axis=1)
        gates = (jnp.dot(qs, wih_ref[...], preferred_element_type=f32)
                 + bih_ref[...]
                 + jnp.dot(qp, whh_ref[...], preferred_element_type=f32)
                 + bhh_ref[...])
        i_ = jax.nn.sigmoid(gates[:, :IN2])
        ff = jax.nn.sigmoid(gates[:, IN2:2 * IN2])
        g_ = jnp.tanh(gates[:, 2 * IN2:3 * IN2])
        o_ = jax.nn.sigmoid(gates[:, 3 * IN2:])
        lcn = ff * lc_ref[...] + i_ * g_
        q = o_ * jnp.tanh(lcn)
        qtab_ref[...] = jnp.concatenate(
            [q, jnp.zeros((ACCM - NMOL, IN2), f32)], axis=0)
        lco_ref[...] = lcn

    return pl.pallas_call(
        body,
        out_shape=[jax.ShapeDtypeStruct((ACCM, IN2), f32),
                   jax.ShapeDtypeStruct((NMOL, IN2), f32)],
    )(wa0, wa1, qprev, lc, wih_t, whh_t, bih, bhh)


def _ew(x, x0, qg):
    def body(x_ref, x0_ref, qg_ref, out_ref):
        nag = jnp.concatenate([x_ref[...], x0_ref[...]], axis=1)
        e = jnp.sum(nag * qg_ref[...], axis=1, keepdims=True)
        gmax = jnp.max(e)
        ex = jnp.exp(e - gmax)
        out_ref[...] = jnp.concatenate(
            [ex * nag, ex, jnp.zeros((NPAD, WD - IN2 - 1), f32)], axis=1)

    return pl.pallas_call(
        body,
        out_shape=jax.ShapeDtypeStruct((NPAD, WD), f32),
    )(x, x0, qg)


def _final(wa0, wa1, qprev, wsp, bsp, prelu, rxn_bc):
    def body(wa0_ref, wa1_ref, q_ref, wsp_ref, bsp_ref, pr_ref, rxn_ref,
             out_ref):
        acc = wa0_ref[...] + wa1_ref[...]
        accn = acc[:NMOL]
        ro = accn[:, :IN2] / accn[:, IN2:IN2 + 1]
        qs = jnp.concatenate([q_ref[...], ro], axis=1)
        rf = jnp.dot(qs, wsp_ref[...], preferred_element_type=f32) + bsp_ref[...]
        rf = jnp.where(rf >= jnp.float32(0.0), rf, pr_ref[...] * rf)
        rids = rxn_ref[...][0:1, :]
        rr = lax.broadcasted_iota(jnp.int32, (NRXN, NMOL), 0).astype(f32)
        mask = jnp.where(rids == rr, jnp.float32(1.0), jnp.float32(0.0))
        out_ref[...] = jnp.dot(mask, rf, preferred_element_type=f32)

    return pl.pallas_call(
        body,
        out_shape=jax.ShapeDtypeStruct((NRXN, DH), f32),
    )(wa0, wa1, qprev, wsp, bsp, prelu, rxn_bc)


_J = np.arange(H * H)
_R1_NP = (_J[None, :] // H == np.arange(H)[:, None]).astype(np.float32)
_R2_NP = (_J[None, :] % H == np.arange(DE)[:, None]).astype(np.float32)


def kernel(node_attr, edge_attr, W_proj, b_proj, W_bond, b_bond, b_nn,
           gru_Wih, gru_Whh, gru_bih, gru_bhh,
           lstm_Wih, lstm_Whh, lstm_bih, lstm_bhh,
           W_sp, b_sp, prelu_a,
           edge_index, mol_ids, rxn_ids):
    src = edge_index[0].astype(jnp.int32)
    dst = edge_index[1].astype(jnp.int32)
    epad = jnp.full((EPAD - E,), N, jnp.int32)

    def ilv(a):
        return (a.reshape(EPAD // _EB, 8, _EB // 8)
                .transpose(0, 2, 1).reshape(NW, 40, 128))

    src_p = ilv(jnp.concatenate([src, epad]))
    dst_p = ilv(jnp.concatenate([dst, epad]))
    mol32 = mol_ids.astype(jnp.int32)
    mol_p = jnp.concatenate(
        [mol32, jnp.full((NPAD - N,), NMOL, jnp.int32)]).reshape(NW, 4, 80)
    ea = edge_attr.astype(f32)

    w2 = W_bond.astype(f32).reshape(DE, H, H).transpose(1, 0, 2).reshape(H * H, H)
    bm = b_bond.astype(f32).reshape(H, H)
    bp = b_proj.astype(f32).reshape(1, H)
    bnn = b_nn.astype(f32).reshape(1, H)
    g_wih_t = gru_Wih.astype(f32).T
    g_whh_t = gru_Whh.astype(f32).T
    g_bih = gru_bih.astype(f32).reshape(1, 3 * H)
    g_bhh = gru_bhh.astype(f32).reshape(1, 3 * H)
    l_wih_t = lstm_Wih.astype(f32).T
    l_whh_t = lstm_Whh.astype(f32).T
    l_bih = lstm_bih.astype(f32).reshape(1, 4 * IN2)
    l_bhh = lstm_bhh.astype(f32).reshape(1, 4 * IN2)
    wsp = W_sp.astype(f32)
    bsp = b_sp.astype(f32).reshape(1, DH)
    prelu = prelu_a.astype(f32).reshape(1, 1)
    rxn_bc = jnp.broadcast_to(
        rxn_ids.astype(f32).reshape(1, NMOL), (8, NMOL))

    zeros_n = jnp.zeros((NPAD, H), f32)
    zeros_m = jnp.zeros((ACCM, WD), f32)
    r1 = jnp.asarray(_R1_NP)
    r2 = jnp.asarray(_R2_NP)

    x0 = _proj(node_attr.astype(f32), W_proj.astype(f32), bp)
    x = x0

    for _ in range(T):
        xs3 = _sc_gather(x, src_p, NPAD, H, 40, 128, pack128=True)
        msg128 = _msg(ea, xs3.reshape(EPAD * H // 128, 128), r1, r2, w2, bm)
        acc2 = _sc_scatter_add(msg128.reshape(NW, EPAD // NW, H), dst_p,
                               zeros_n, NPAD, H, 40, 128, pack128=True)
        x = _gru(acc2[0], acc2[1], x, bnn, g_wih_t, g_whh_t, g_bih, g_bhh)

    qprev = jnp.zeros((NMOL, IN2), f32)
    lc = jnp.zeros((NMOL, IN2), f32)
    wa0 = zeros_m
    wa1 = zeros_m
    for i in range(ITERS):
        qtab, lc = _s2s_update(i == 0, wa0, wa1, qprev, lc,
                               l_wih_t, l_whh_t, l_bih, l_bhh)
        qprev = qtab[:NMOL]
        qg = _sc_gather(qtab, mol_p, ACCM, IN2, 4, 80)
        w = _ew(x, x0, qg)
        qg = _sc_gather(qtab, mol_p, ACCM, IN2, 4, 80)
        w = _ew(x, x0, qg)
        wacc = _sc_scatter_add(w, mol_p, zeros_m, ACCM, WD, 4, 80)
        wa0 = wacc[0]
        wa1 = wacc[1]

    return _final(wa0, wa1, qprev, wsp, bsp, prelu, rxn_bc)


# speedup vs baseline: 1.1602x; 1.0108x over previous
import functools

import numpy as np
import jax
import jax.numpy as jnp
from jax import lax
from jax.experimental import pallas as pl
from jax.experimental.pallas import tpu as pltpu
from jax.experimental.pallas import tpu_sc as plsc

N = 10000; E = 160000; DN = 128; DE = 16; H = 16
T = 3; ITERS = 3; NMOL = 1000; NRXN = 250; IN2 = 32; DH = 128
NC = 2; NS = 16; NW = NC * NS
NPAD = 10240
EPAD = 163840
ACCM = 1008
WD = 48
f32 = jnp.float32


def _sc_gather(table, idx3, trows, d, c_chunks, k_chunk, pack128=False):
    per = c_chunks * k_chunk
    mesh = plsc.VectorSubcoreMesh(core_axis_name="c", subcore_axis_name="s")
    if pack128:
        out_t = jax.ShapeDtypeStruct((NW, per, d), f32)
    else:
        out_t = jax.ShapeDtypeStruct((NW * per, d), f32)

    @functools.partial(
        pl.kernel,
        out_type=out_t,
        mesh=mesh,
        compiler_params=pltpu.CompilerParams(use_tc_tiling_on_sc=False),
        scratch_types=[
            pltpu.VMEM((c_chunks, k_chunk), jnp.int32),
            pltpu.VMEM((per, d), f32),
            pltpu.SemaphoreType.DMA,
        ],
    )
    def k(table_hbm, idx_hbm, out_hbm, idx_v, rows_v, sem):
        wid = (lax.axis_index("c") * np.int32(NS)
               + lax.axis_index("s")).astype(jnp.int32)
        pltpu.sync_copy(idx_hbm.at[wid], idx_v)

        wsz = 20 if c_chunks % 20 == 0 else min(8, c_chunks)

        def wave(wv, carry):
            descs = []
            for j in range(wsz):
                c = wv * np.int32(wsz) + np.int32(j)
                descs.append(pltpu.async_copy(
                    table_hbm.at[idx_v.at[c]],
                    rows_v.at[pl.ds(c * np.int32(k_chunk), k_chunk)],
                    sem,
                ))
            for dsc in descs:
                dsc.wait()
            return carry

        lax.fori_loop(jnp.int32(0), jnp.int32(c_chunks // wsz), wave,
                      jnp.int32(0))

        if pack128:
            pltpu.sync_copy(rows_v, out_hbm.at[wid])
        else:
            pltpu.sync_copy(rows_v,
                            out_hbm.at[pl.ds(wid * np.int32(per), per)])

    return k(table, idx3)


def _sc_scatter_add(vals, idx3, zeros, arows, d, c_chunks, k_chunk,
                    pack128=False):
    per = c_chunks * k_chunk
    rps = arows // NS
    prow = per * d // 128
    mesh = plsc.VectorSubcoreMesh(core_axis_name="c", subcore_axis_name="s")

    @functools.partial(
        pl.kernel,
        out_type=jax.ShapeDtypeStruct((NC, arows, d), f32),
        mesh=mesh,
        compiler_params=pltpu.CompilerParams(use_tc_tiling_on_sc=False),
        scratch_types=[
            pltpu.VMEM((c_chunks, k_chunk), jnp.int32),
            pltpu.VMEM((per, d), f32),
            pltpu.VMEM_SHARED((arows, d), f32),
            pltpu.SemaphoreType.DMA,
        ],
    )
    def k(vals_hbm, idx_hbm, zeros_hbm, out_hbm, idx_v, vals_v, acc, sem):
        cid = lax.axis_index("c").astype(jnp.int32)
        sid = lax.axis_index("s").astype(jnp.int32)
        wid = cid * np.int32(NS) + sid
        pltpu.sync_copy(zeros_hbm.at[pl.ds(sid * np.int32(rps), rps)],
                        acc.at[pl.ds(sid * np.int32(rps), rps)])
        pltpu.sync_copy(idx_hbm.at[wid], idx_v)
        if pack128:
            pltpu.sync_copy(vals_hbm.at[wid], vals_v)
        else:
            pltpu.sync_copy(vals_hbm.at[pl.ds(wid * np.int32(per), per)],
                            vals_v)
        plsc.subcore_barrier()

        wsz = 20 if c_chunks % 20 == 0 else min(8, c_chunks)

        def wave(wv, carry):
            descs = []
            for j in range(wsz):
                c = wv * np.int32(wsz) + np.int32(j)
                descs.append(pltpu.async_copy(
                    vals_v.at[pl.ds(c * np.int32(k_chunk), k_chunk)],
                    acc.at[idx_v.at[c]], sem, add=True))
            for dsc in descs:
                dsc.wait()
            return carry

        lax.fori_loop(jnp.int32(0), jnp.int32(c_chunks // wsz), wave,
                      jnp.int32(0))

        plsc.subcore_barrier()
        pltpu.sync_copy(acc.at[pl.ds(sid * np.int32(rps), rps)],
                        out_hbm.at[cid, pl.ds(sid * np.int32(rps), rps)])

    return k(vals, idx3, zeros)


def _proj(na, wp, bp):
    def body(na_ref, wp_ref, bp_ref, out_ref):
        y = jnp.maximum(
            jnp.dot(na_ref[...], wp_ref[...], preferred_element_type=f32)
            + bp_ref[...], 0.0)
        out_ref[...] = jnp.concatenate(
            [y, jnp.zeros((NPAD - N, H), f32)], axis=0)

    return pl.pallas_call(
        body,
        out_shape=jax.ShapeDtypeStruct((NPAD, H), f32),
    )(na, wp, bp)


_EB = 16384


def _msg(ea, xs128, r1, r2, w2, bm):
    pb = _EB * H // 128
    G = _EB // 8

    def body(ea_ref, xs_ref, r1_ref, r2_ref, w2_ref, bm_ref, out_ref):
        r1_ = r1_ref[...]
        r2_ = r2_ref[...]
        w2_ = w2_ref[...]
        bm_ = bm_ref[...]
        for j in range(8):
            xj = xs_ref[:, j * H:(j + 1) * H]
            eaj = ea_ref[j * G:(j + 1) * G, :]
            opj = (jnp.dot(xj, r1_, preferred_element_type=f32)
                   * jnp.dot(eaj, r2_, preferred_element_type=f32))
            out_ref[:, j * H:(j + 1) * H] = (
                jnp.dot(opj, w2_, preferred_element_type=f32)
                + jnp.dot(xj, bm_, preferred_element_type=f32))

    z32 = np.int32(0)
    nb = EPAD // _EB
    return pl.pallas_call(
        body,
        grid=(nb,),
        in_specs=[
            pl.BlockSpec((_EB, DE), lambda i: (i, z32)),
            pl.BlockSpec((pb, 128), lambda i: (i, z32)),
            pl.BlockSpec((H, H * H), lambda i: (z32, z32)),
            pl.BlockSpec((DE, H * H), lambda i: (z32, z32)),
            pl.BlockSpec((H * H, H), lambda i: (z32, z32)),
            pl.BlockSpec((H, H), lambda i: (z32, z32)),
        ],
        out_specs=pl.BlockSpec((pb, 128), lambda i: (i, z32)),
        out_shape=jax.ShapeDtypeStruct((EPAD * H // 128, 128), f32),
    )(ea, xs128, r1, r2, w2, bm)


def _gru(a0, a1, h, bnn, wih_t, whh_t, bih, bhh):
    def body(a0_ref, a1_ref, h_ref, bnn_ref, wih_ref, whh_ref, bih_ref,
             bhh_ref, out_ref):
        xa = jnp.maximum(a0_ref[...] + a1_ref[...] + bnn_ref[...], 0.0)
        h_ = h_ref[...]
        gi = jnp.dot(xa, wih_ref[...], preferred_element_type=f32) + bih_ref[...]
        gh = jnp.dot(h_, whh_ref[...], preferred_element_type=f32) + bhh_ref[...]
        r = jax.nn.sigmoid(gi[:, :H] + gh[:, :H])
        z = jax.nn.sigmoid(gi[:, H:2 * H] + gh[:, H:2 * H])
        n_ = jnp.tanh(gi[:, 2 * H:] + r * gh[:, 2 * H:])
        hn = (1.0 - z) * n_ + z * h_
        rows = lax.broadcasted_iota(jnp.int32, (NPAD, H), 0)
        out_ref[...] = jnp.where(rows < N, hn, 0.0)

    return pl.pallas_call(
        body,
        out_shape=jax.ShapeDtypeStruct((NPAD, H), f32),
    )(a0, a1, h, bnn, wih_t, whh_t, bih, bhh)


def _s2s_update(is_first, wa0, wa1, qprev, lc, wih_t, whh_t, bih, bhh):
    def body(wa0_ref, wa1_ref, q_ref, lc_ref, wih_ref, whh_ref, bih_ref,
             bhh_ref, qtab_ref, lco_ref):
        qp = q_ref[...]
        if is_first:
            qs = jnp.zeros((NMOL, 2 * IN2), f32)
        else:
            acc = wa0_ref[...] + wa1_ref[...]
            accn = acc[:NMOL]
            ro = accn[:, :IN2] / accn[:, IN2:IN2 + 1]
            qs = jnp.concatenate([qp, ro], axis=1)
        gates = (jnp.dot(qs, wih_ref[...], preferred_element_type=f32)
                 + bih_ref[...]
                 + jnp.dot(qp, whh_ref[...], preferred_element_type=f32)
                 + bhh_ref[...])
        i_ = jax.nn.sigmoid(gates[:, :IN2])
        ff = jax.nn.sigmoid(gates[:, IN2:2 * IN2])
        g_ = jnp.tanh(gates[:, 2 * IN2:3 * IN2])
        o_ = jax.nn.sigmoid(gates[:, 3 * IN2:])
        lcn = ff * lc_ref[...] + i_ * g_
        q = o_ * jnp.tanh(lcn)
        qtab_ref[...] = jnp.concatenate(
            [q, jnp.zeros((ACCM - NMOL, IN2), f32)], axis=0)
        lco_ref[...] = lcn

    return pl.pallas_call(
        body,
        out_shape=[jax.ShapeDtypeStruct((ACCM, IN2), f32),
                   jax.ShapeDtypeStruct((NMOL, IN2), f32)],
    )(wa0, wa1, qprev, lc, wih_t, whh_t, bih, bhh)


def _ew(x, x0, qg):
    def body(x_ref, x0_ref, qg_ref, out_ref):
        nag = jnp.concatenate([x_ref[...], x0_ref[...]], axis=1)
        e = jnp.sum(nag * qg_ref[...], axis=1, keepdims=True)
        gmax = jnp.max(e)
        ex = jnp.exp(e - gmax)
        out_ref[...] = jnp.concatenate(
            [ex * nag, ex, jnp.zeros((NPAD, WD - IN2 - 1), f32)], axis=1)

    return pl.pallas_call(
        body,
        out_shape=jax.ShapeDtypeStruct((NPAD, WD), f32),
    )(x, x0, qg)


def _final(wa0, wa1, qprev, wsp, bsp, prelu, rxn_bc):
    def body(wa0_ref, wa1_ref, q_ref, wsp_ref, bsp_ref, pr_ref, rxn_ref,
             out_ref):
        acc = wa0_ref[...] + wa1_ref[...]
        accn = acc[:NMOL]
        ro = accn[:, :IN2] / accn[:, IN2:IN2 + 1]
        qs = jnp.concatenate([q_ref[...], ro], axis=1)
        rf = jnp.dot(qs, wsp_ref[...], preferred_element_type=f32) + bsp_ref[...]
        rf = jnp.where(rf >= jnp.float32(0.0), rf, pr_ref[...] * rf)
        rids = rxn_ref[...][0:1, :]
        rr = lax.broadcasted_iota(jnp.int32, (NRXN, NMOL), 0).astype(f32)
        mask = jnp.where(rids == rr, jnp.float32(1.0), jnp.float32(0.0))
        out_ref[...] = jnp.dot(mask, rf, preferred_element_type=f32)

    return pl.pallas_call(
        body,
        out_shape=jax.ShapeDtypeStruct((NRXN, DH), f32),
    )(wa0, wa1, qprev, wsp, bsp, prelu, rxn_bc)


_J = np.arange(H * H)
_R1_NP = (_J[None, :] // H == np.arange(H)[:, None]).astype(np.float32)
_R2_NP = (_J[None, :] % H == np.arange(DE)[:, None]).astype(np.float32)


def kernel(node_attr, edge_attr, W_proj, b_proj, W_bond, b_bond, b_nn,
           gru_Wih, gru_Whh, gru_bih, gru_bhh,
           lstm_Wih, lstm_Whh, lstm_bih, lstm_bhh,
           W_sp, b_sp, prelu_a,
           edge_index, mol_ids, rxn_ids):
    src = edge_index[0].astype(jnp.int32)
    dst = edge_index[1].astype(jnp.int32)
    epad = jnp.full((EPAD - E,), N, jnp.int32)

    def ilv(a):
        return (a.reshape(EPAD // _EB, 8, _EB // 8)
                .transpose(0, 2, 1).reshape(NW, 40, 128))

    src_p = ilv(jnp.concatenate([src, epad]))
    dst_p = ilv(jnp.concatenate([dst, epad]))
    mol32 = mol_ids.astype(jnp.int32)
    mol_p = jnp.concatenate(
        [mol32, jnp.full((NPAD - N,), NMOL, jnp.int32)]).reshape(NW, 4, 80)
    ea = edge_attr.astype(f32)

    w2 = W_bond.astype(f32).reshape(DE, H, H).transpose(1, 0, 2).reshape(H * H, H)
    bm = b_bond.astype(f32).reshape(H, H)
    bp = b_proj.astype(f32).reshape(1, H)
    bnn = b_nn.astype(f32).reshape(1, H)
    g_wih_t = gru_Wih.astype(f32).T
    g_whh_t = gru_Whh.astype(f32).T
    g_bih = gru_bih.astype(f32).reshape(1, 3 * H)
    g_bhh = gru_bhh.astype(f32).reshape(1, 3 * H)
    l_wih_t = lstm_Wih.astype(f32).T
    l_whh_t = lstm_Whh.astype(f32).T
    l_bih = lstm_bih.astype(f32).reshape(1, 4 * IN2)
    l_bhh = lstm_bhh.astype(f32).reshape(1, 4 * IN2)
    wsp = W_sp.astype(f32)
    bsp = b_sp.astype(f32).reshape(1, DH)
    prelu = prelu_a.astype(f32).reshape(1, 1)
    rxn_bc = jnp.broadcast_to(
        rxn_ids.astype(f32).reshape(1, NMOL), (8, NMOL))

    zeros_n = jnp.zeros((NPAD, H), f32)
    zeros_m = jnp.zeros((ACCM, WD), f32)
    r1 = jnp.asarray(_R1_NP)
    r2 = jnp.asarray(_R2_NP)

    x0 = _proj(node_attr.astype(f32), W_proj.astype(f32), bp)
    x = x0

    for _ in range(T):
        xs3 = _sc_gather(x, src_p, NPAD, H, 40, 128, pack128=True)
        msg128 = _msg(ea, xs3.reshape(EPAD * H // 128, 128), r1, r2, w2, bm)
        acc2 = _sc_scatter_add(msg128.reshape(NW, EPAD // NW, H), dst_p,
                               zeros_n, NPAD, H, 40, 128, pack128=True)
        x = _gru(acc2[0], acc2[1], x, bnn, g_wih_t, g_whh_t, g_bih, g_bhh)

    qprev = jnp.zeros((NMOL, IN2), f32)
    lc = jnp.zeros((NMOL, IN2), f32)
    wa0 = zeros_m
    wa1 = zeros_m
    for i in range(ITERS):
        qtab, lc = _s2s_update(i == 0, wa0, wa1, qprev, lc,
                               l_wih_t, l_whh_t, l_bih, l_bhh)
        qprev = qtab[:NMOL]
        qg = _sc_gather(qtab, mol_p, ACCM, IN2, 4, 80)
        w = _ew(x, x0, qg)
        qg = _sc_gather(qtab, mol_p, ACCM, IN2, 4, 80)
        w = _ew(x, x0, qg)
        wacc = _sc_scatter_add(w, mol_p, zeros_m, ACCM, WD, 4, 80)
        wa0 = wacc[0]
        wa1 = wacc[1]

    return _final(wa0, wa1, qprev, wsp, bsp, prelu, rxn_bc)
